# baseline (speedup 1.0000x reference)
import functools

import jax
import jax.numpy as jnp
from jax import lax
from jax.experimental import pallas as pl
from jax.experimental.pallas import tpu as pltpu
from jax.experimental.pallas import tpu_sc as plsc

_NC = 2
_NS = 16
_NW = _NC * _NS
_NBUF = 8


@functools.lru_cache(maxsize=None)
def _make_gather(N, S, V, D):
    n_per_w = N // _NW
    mesh = plsc.VectorSubcoreMesh(core_axis_name="c", subcore_axis_name="s")

    @functools.partial(
        pl.kernel,
        mesh=mesh,
        out_type=jax.ShapeDtypeStruct((N, S, D), jnp.float32),
        scratch_types=[
            pltpu.VMEM((n_per_w, S), jnp.int32),
            pltpu.VMEM((_NBUF, S, D), jnp.float32),
            pltpu.SemaphoreType.DMA,
            pltpu.SemaphoreType.DMA,
        ],
        compiler_params=pltpu.CompilerParams(use_tc_tiling_on_sc=True),
    )
    def gather_kernel(idx_hbm, table_hbm, out_hbm, idx_v, bufs, gsem, wsem):
        wid = lax.axis_index("s") * _NC + lax.axis_index("c")
        base = wid * n_per_w
        pltpu.sync_copy(idx_hbm.at[pl.ds(base, n_per_w)], idx_v)

        def gather_copy(c):
            return pltpu.make_async_copy(
                table_hbm.at[idx_v.at[c]], bufs.at[c % _NBUF], gsem)

        def write_copy(c):
            return pltpu.make_async_copy(
                bufs.at[c % _NBUF], out_hbm.at[base + c], wsem)

        for c in range(_NBUF - 1):
            gather_copy(c).start()

        def body(c, carry):
            gather_copy(c).wait()
            write_copy(c).start()

            @pl.when(c + _NBUF - 1 < n_per_w)
            def _():
                @pl.when(c >= 1)
                def _():
                    write_copy(c).wait()

                gather_copy(c + _NBUF - 1).start()

            return carry

        lax.fori_loop(0, n_per_w, body, 0)
        for _ in range(_NBUF):
            write_copy(0).wait()

    return gather_kernel


def kernel(resids_positional_encoded, table):
    idx = resids_positional_encoded.astype(jnp.int32)
    n, s = idx.shape
    V, D = table.shape
    return _make_gather(n, s, V, D)(idx, table)


# speedup vs baseline: 1.8004x; 1.8004x over previous
import functools

import jax
import jax.numpy as jnp
from jax import lax
from jax.experimental import pallas as pl
from jax.experimental.pallas import tpu as pltpu
from jax.experimental.pallas import tpu_sc as plsc

_NC = 2
_NS = 16
_NW = _NC * _NS
_NBUF = 6


@functools.lru_cache(maxsize=None)
def _make_gather(N, S, V, D):
    n_per_w = N // _NW

    mesh = plsc.VectorSubcoreMesh(core_axis_name="c", subcore_axis_name="s")

    @functools.partial(
        pl.kernel,
        mesh=mesh,
        out_type=jax.ShapeDtypeStruct((S, N, D), jnp.float32),
        scratch_types=[
            pltpu.VMEM((S, n_per_w), jnp.int32),
            pltpu.VMEM((_NBUF, n_per_w, D), jnp.float32),
            pltpu.SemaphoreType.DMA,
            pltpu.SemaphoreType.DMA,
        ],
        compiler_params=pltpu.CompilerParams(use_tc_tiling_on_sc=True),
    )
    def gather_kernel(idxt_hbm, table_hbm, out_hbm, idx_v, bufs, gsem, wsem):
        wid = lax.axis_index("s") * _NC + lax.axis_index("c")
        base = wid * n_per_w
        pltpu.sync_copy(idxt_hbm.at[:, pl.ds(base, n_per_w)], idx_v)

        def gather_copy(c):
            return pltpu.make_async_copy(
                table_hbm.at[idx_v.at[c]], bufs.at[c % _NBUF], gsem)

        def write_copy(c):
            return pltpu.make_async_copy(
                bufs.at[c % _NBUF], out_hbm.at[c, pl.ds(base, n_per_w)], wsem)

        for c in range(_NBUF - 1):
            gather_copy(c).start()

        def body(c, carry):
            gather_copy(c).wait()
            write_copy(c).start()

            @pl.when(c + _NBUF - 1 < S)
            def _():
                @pl.when(c >= 1)
                def _():
                    write_copy(c).wait()

                gather_copy(c + _NBUF - 1).start()

            return carry

        lax.fori_loop(0, S, body, 0)
        for _ in range(_NBUF):
            write_copy(0).wait()

    return gather_kernel


def kernel(resids_positional_encoded, table):
    idx = resids_positional_encoded.astype(jnp.int32)
    n, s = idx.shape
    V, D = table.shape
    out = _make_gather(n, s, V, D)(idx.T, table)
    return out.transpose(1, 0, 2)


# speedup vs baseline: 1.8017x; 1.0007x over previous
import functools

import jax
import jax.numpy as jnp
from jax import lax
from jax.experimental import pallas as pl
from jax.experimental.pallas import tpu as pltpu
from jax.experimental.pallas import tpu_sc as plsc

_NC = 2
_NS = 16
_NW = _NC * _NS
_NBUF = 7


@functools.lru_cache(maxsize=None)
def _make_gather(N, S, V, D):
    n_per_w = N // _NW

    mesh = plsc.VectorSubcoreMesh(core_axis_name="c", subcore_axis_name="s")

    @functools.partial(
        pl.kernel,
        mesh=mesh,
        out_type=jax.ShapeDtypeStruct((S, N, D), jnp.float32),
        scratch_types=[
            pltpu.VMEM((S, n_per_w), jnp.int32),
            pltpu.VMEM((_NBUF, n_per_w, D), jnp.float32),
            pltpu.SemaphoreType.DMA,
            pltpu.SemaphoreType.DMA,
        ],
        compiler_params=pltpu.CompilerParams(use_tc_tiling_on_sc=True),
    )
    def gather_kernel(idxt_hbm, table_hbm, out_hbm, idx_v, bufs, gsem, wsem):
        wid = lax.axis_index("s") * _NC + lax.axis_index("c")
        base = wid * n_per_w
        pltpu.sync_copy(idxt_hbm.at[:, pl.ds(base, n_per_w)], idx_v)

        def gather_copy(c):
            return pltpu.make_async_copy(
                table_hbm.at[idx_v.at[c]], bufs.at[c % _NBUF], gsem)

        def write_copy(c):
            return pltpu.make_async_copy(
                bufs.at[c % _NBUF], out_hbm.at[c, pl.ds(base, n_per_w)], wsem)

        for c in range(_NBUF - 1):
            gather_copy(c).start()

        def body(c, carry):
            gather_copy(c).wait()
            write_copy(c).start()

            @pl.when(c + _NBUF - 1 < S)
            def _():
                @pl.when(c >= 1)
                def _():
                    write_copy(c).wait()

                gather_copy(c + _NBUF - 1).start()

            return carry

        lax.fori_loop(0, S, body, 0)
        for _ in range(_NBUF):
            write_copy(0).wait()

    return gather_kernel


def kernel(resids_positional_encoded, table):
    idx = resids_positional_encoded.astype(jnp.int32)
    n, s = idx.shape
    V, D = table.shape
    out = _make_gather(n, s, V, D)(idx.T, table)
    return out.transpose(1, 0, 2)
